# initial kernel scaffold (unmeasured)
import jax
import jax.numpy as jnp
from jax import lax
from jax.experimental import pallas as pl
from jax.experimental.pallas import tpu as pltpu

N_DEV = 4


def _ag_body(x_ref, out_ref, copy_sem, send_sems, recv_sems):
    my = lax.axis_index("i")
    right = lax.rem(my + 1, N_DEV)
    m_per = x_ref.shape[0]

    local = pltpu.make_async_copy(
        x_ref, out_ref.at[pl.ds(my * m_per, m_per), :], copy_sem
    )
    local.start()

    for h in range(N_DEV - 1):
        origin = lax.rem(my - h + N_DEV, N_DEV)
        if h == 0:
            src = x_ref
        else:
            src = out_ref.at[pl.ds(origin * m_per, m_per), :]
        rdma = pltpu.make_async_remote_copy(
            src_ref=src,
            dst_ref=out_ref.at[pl.ds(origin * m_per, m_per), :],
            send_sem=send_sems.at[h],
            recv_sem=recv_sems.at[h],
            device_id=(right,),
            device_id_type=pl.DeviceIdType.MESH,
        )
        rdma.start()
        rdma.wait()

    local.wait()


def _mm_body(x_ref, w_ref, o_ref):
    acc = jnp.dot(
        x_ref[...].astype(jnp.bfloat16),
        w_ref[...].astype(jnp.bfloat16),
        preferred_element_type=jnp.float32,
    )
    o_ref[...] = acc * jax.nn.sigmoid(acc)


def kernel(x, w_mat):
    m_per, k = x.shape
    _, n_per = w_mat.shape
    m = N_DEV * m_per

    full_x = pl.pallas_call(
        _ag_body,
        out_shape=jax.ShapeDtypeStruct((m, k), x.dtype),
        in_specs=[pl.BlockSpec(memory_space=pltpu.ANY)],
        out_specs=pl.BlockSpec(memory_space=pltpu.ANY),
        scratch_shapes=[
            pltpu.SemaphoreType.DMA,
            pltpu.SemaphoreType.DMA((N_DEV - 1,)),
            pltpu.SemaphoreType.DMA((N_DEV - 1,)),
        ],
        compiler_params=pltpu.CompilerParams(collective_id=0),
    )(x)

    bm, bn = 512, 512
    out = pl.pallas_call(
        _mm_body,
        out_shape=jax.ShapeDtypeStruct((m, n_per), jnp.float32),
        grid=(m // bm, n_per // bn),
        in_specs=[
            pl.BlockSpec((bm, k), lambda i, j: (i, 0)),
            pl.BlockSpec((k, bn), lambda i, j: (0, j)),
        ],
        out_specs=pl.BlockSpec((bm, bn), lambda i, j: (i, j)),
    )(full_x, w_mat)
    return out


# baseline (device time: 687556 ns/iter reference)
import jax
import jax.numpy as jnp
from jax import lax
from jax.experimental import pallas as pl
from jax.experimental.pallas import tpu as pltpu

N_DEV = 4


def _ag_body(x_ref, out_ref, copy_sem, send_sems, recv_sems):
    my = lax.axis_index("i")
    right = lax.rem(my + 1, N_DEV)
    m_per = x_ref.shape[0]

    local = pltpu.make_async_copy(
        x_ref, out_ref.at[pl.ds(my * m_per, m_per), :], copy_sem
    )
    local.start()

    for h in range(N_DEV - 1):
        origin = lax.rem(my - h + N_DEV, N_DEV)
        if h == 0:
            src = x_ref
        else:
            src = out_ref.at[pl.ds(origin * m_per, m_per), :]
        rdma = pltpu.make_async_remote_copy(
            src_ref=src,
            dst_ref=out_ref.at[pl.ds(origin * m_per, m_per), :],
            send_sem=send_sems.at[h],
            recv_sem=recv_sems.at[h],
            device_id=(right,),
            device_id_type=pl.DeviceIdType.MESH,
        )
        rdma.start()
        rdma.wait()

    local.wait()


def _mm_body(x_ref, w_ref, o_ref):
    acc = jnp.dot(
        x_ref[...].astype(jnp.bfloat16),
        w_ref[...].astype(jnp.bfloat16),
        preferred_element_type=jnp.float32,
    )
    o_ref[...] = acc * jax.nn.sigmoid(acc)


def kernel(x, w_mat):
    m_per, k = x.shape
    _, n_per = w_mat.shape
    m = N_DEV * m_per

    full_x = pl.pallas_call(
        _ag_body,
        out_shape=jax.ShapeDtypeStruct((m, k), x.dtype),
        in_specs=[pl.BlockSpec(memory_space=pltpu.MemorySpace.HBM)],
        out_specs=pl.BlockSpec(memory_space=pltpu.MemorySpace.HBM),
        scratch_shapes=[
            pltpu.SemaphoreType.DMA,
            pltpu.SemaphoreType.DMA((N_DEV - 1,)),
            pltpu.SemaphoreType.DMA((N_DEV - 1,)),
        ],
    )(x)

    bm, bn = 512, 512
    out = pl.pallas_call(
        _mm_body,
        out_shape=jax.ShapeDtypeStruct((m, n_per), jnp.float32),
        grid=(m // bm, n_per // bn),
        in_specs=[
            pl.BlockSpec((bm, k), lambda i, j: (i, 0)),
            pl.BlockSpec((k, bn), lambda i, j: (0, j)),
        ],
        out_specs=pl.BlockSpec((bm, bn), lambda i, j: (i, j)),
        compiler_params=pltpu.CompilerParams(
            vmem_limit_bytes=56 * 1024 * 1024
        ),
    )(full_x, w_mat)
    return out
